# Initial kernel scaffold; baseline (speedup 1.0000x reference)
import jax
import jax.numpy as jnp
from jax.experimental import pallas as pl


def kernel(x, knots, a, b, c, d):
    raise NotImplementedError("write your pallas kernel here")


# speedup vs baseline: 102.2690x; 102.2690x over previous
import functools
import math

import jax
import jax.numpy as jnp
from jax import lax
from jax.experimental import pallas as pl
from jax.experimental.pallas import tpu as pltpu
from jax.experimental.pallas import tpu_sc as plsc

L = 16
NC, NS = 2, 16
NW = NC * NS
CHUNK = 4000


def _search_and_eval(x, s_ref, a_ref, b_ref, c_ref, d_ref, bits):
    lo = jnp.zeros((L,), jnp.int32)
    for bit in bits:
        t = lo + bit
        v = plsc.load_gather(s_ref, [t])
        lo = jnp.where(v < x, t, lo)
    av = plsc.load_gather(a_ref, [lo])
    bv = plsc.load_gather(b_ref, [lo])
    cv = plsc.load_gather(c_ref, [lo])
    dv = plsc.load_gather(d_ref, [lo])
    return ((dv * x + cv) * x + bv) * x + av


def _make_sc_kernel(n, k):
    assert n % CHUNK == 0 and CHUNK % L == 0
    n_chunks = n // CHUNK
    chunks_per_worker = -(-n_chunks // NW)
    n_vec = CHUNK // L
    top_bit = 1 << (math.ceil(math.log2(k)) - 1)
    bits = []
    b = top_bit
    while b:
        bits.append(b)
        b >>= 1

    mesh = plsc.VectorSubcoreMesh(core_axis_name="c", subcore_axis_name="s")

    @functools.partial(
        pl.kernel,
        out_type=jax.ShapeDtypeStruct((n,), jnp.float32),
        mesh=mesh,
        compiler_params=pltpu.CompilerParams(needs_layout_passes=False),
        scratch_types=[
            pltpu.VMEM((2 * CHUNK,), jnp.float32),
            pltpu.VMEM((CHUNK,), jnp.float32),
            pltpu.VMEM((2 * top_bit,), jnp.float32),
            pltpu.VMEM((2 * top_bit,), jnp.float32),
        ] + [pltpu.VMEM((k,), jnp.float32) for _ in range(8)],
    )
    def sc_kernel(x_hbm, s0_hbm, s1_hbm, a0_hbm, b0_hbm, c0_hbm, d0_hbm,
                  a1_hbm, b1_hbm, c1_hbm, d1_hbm, out_hbm,
                  x_v, out_v, s0_v, s1_v, a0_v, b0_v, c0_v, d0_v,
                  a1_v, b1_v, c1_v, d1_v):
        wid = lax.axis_index("s") * NC + lax.axis_index("c")

        pltpu.sync_copy(s0_hbm, s0_v)
        pltpu.sync_copy(s1_hbm, s1_v)
        pltpu.sync_copy(a0_hbm, a0_v)
        pltpu.sync_copy(b0_hbm, b0_v)
        pltpu.sync_copy(c0_hbm, c0_v)
        pltpu.sync_copy(d0_hbm, d0_v)
        pltpu.sync_copy(a1_hbm, a1_v)
        pltpu.sync_copy(b1_hbm, b1_v)
        pltpu.sync_copy(c1_hbm, c1_v)
        pltpu.sync_copy(d1_hbm, d1_v)

        iota2 = lax.iota(jnp.int32, L) * 2

        def chunk_body(c, _):
            chunk_id = wid + c * NW

            @pl.when(chunk_id < n_chunks)
            def _():
                base = chunk_id * CHUNK
                pltpu.sync_copy(x_hbm.at[pl.ds(2 * base, 2 * CHUNK)], x_v)

                def vec_body(v, _):
                    flat = v * (2 * L) + iota2
                    x0 = plsc.load_gather(x_v, [flat])
                    x1 = plsc.load_gather(x_v, [flat + 1])
                    p0 = _search_and_eval(x0, s0_v, a0_v, b0_v, c0_v, d0_v,
                                          bits)
                    p1 = _search_and_eval(x1, s1_v, a1_v, b1_v, c1_v, d1_v,
                                          bits)
                    out_v[pl.ds(v * L, L)] = p0 * p1
                    return _

                lax.fori_loop(0, n_vec, vec_body, None)
                pltpu.sync_copy(out_v, out_hbm.at[pl.ds(base, CHUNK)])

            return _

        lax.fori_loop(0, chunks_per_worker, chunk_body, None)

    return sc_kernel


def kernel(x, knots, a, b, c, d):
    n = x.shape[0]
    k = knots.shape[0]
    top_bit = 1 << (math.ceil(math.log2(k)) - 1)
    pad = 2 * top_bit - (k - 1)
    inf = jnp.full((pad,), jnp.inf, jnp.float32)
    s0 = jnp.concatenate([knots[:k - 1, 0], inf])
    s1 = jnp.concatenate([knots[:k - 1, 1], inf])

    def col(t, j):
        return jnp.concatenate([t[:, j], jnp.zeros((1,), jnp.float32)])

    sc = _make_sc_kernel(n, k)
    return sc(x.reshape(-1), s0, s1,
              col(a, 0), col(b, 0), col(c, 0), col(d, 0),
              col(a, 1), col(b, 1), col(c, 1), col(d, 1))


# speedup vs baseline: 112.2702x; 1.0978x over previous
import functools
import math

import jax
import jax.numpy as jnp
from jax import lax
from jax.experimental import pallas as pl
from jax.experimental.pallas import tpu as pltpu
from jax.experimental.pallas import tpu_sc as plsc

L = 16
NC, NS = 2, 16
NW = NC * NS
CHUNK = 4000


def _search_and_eval(x, s_ref, a_ref, b_ref, c_ref, d_ref, bits):
    lo = jnp.zeros((L,), jnp.int32)
    for bit in bits:
        t = lo + bit
        v = plsc.load_gather(s_ref, [t])
        lo = jnp.where(v < x, t, lo)
    av = plsc.load_gather(a_ref, [lo])
    bv = plsc.load_gather(b_ref, [lo])
    cv = plsc.load_gather(c_ref, [lo])
    dv = plsc.load_gather(d_ref, [lo])
    return ((dv * x + cv) * x + bv) * x + av


def _make_sc_kernel(n, k):
    assert n % CHUNK == 0 and CHUNK % L == 0
    n_chunks = n // CHUNK
    chunks_per_worker = -(-n_chunks // NW)
    n_vec = CHUNK // L
    top_bit = 1 << (math.ceil(math.log2(k)) - 1)
    bits = []
    b = top_bit
    while b:
        bits.append(b)
        b >>= 1

    mesh = plsc.VectorSubcoreMesh(core_axis_name="c", subcore_axis_name="s")

    @functools.partial(
        pl.kernel,
        out_type=jax.ShapeDtypeStruct((n,), jnp.float32),
        mesh=mesh,
        compiler_params=pltpu.CompilerParams(needs_layout_passes=False),
        scratch_types=[
            pltpu.VMEM((2 * CHUNK,), jnp.float32),
            pltpu.VMEM((CHUNK,), jnp.float32),
            pltpu.VMEM((2 * top_bit,), jnp.float32),
            pltpu.VMEM((2 * top_bit,), jnp.float32),
        ] + [pltpu.VMEM((k,), jnp.float32) for _ in range(8)],
    )
    def sc_kernel(x_hbm, s0_hbm, s1_hbm, a0_hbm, b0_hbm, c0_hbm, d0_hbm,
                  a1_hbm, b1_hbm, c1_hbm, d1_hbm, out_hbm,
                  x_v, out_v, s0_v, s1_v, a0_v, b0_v, c0_v, d0_v,
                  a1_v, b1_v, c1_v, d1_v):
        wid = lax.axis_index("s") * NC + lax.axis_index("c")

        pltpu.sync_copy(s0_hbm, s0_v)
        pltpu.sync_copy(s1_hbm, s1_v)
        pltpu.sync_copy(a0_hbm, a0_v)
        pltpu.sync_copy(b0_hbm, b0_v)
        pltpu.sync_copy(c0_hbm, c0_v)
        pltpu.sync_copy(d0_hbm, d0_v)
        pltpu.sync_copy(a1_hbm, a1_v)
        pltpu.sync_copy(b1_hbm, b1_v)
        pltpu.sync_copy(c1_hbm, c1_v)
        pltpu.sync_copy(d1_hbm, d1_v)

        iota2 = lax.iota(jnp.int32, L) * 2

        def chunk_body(c, _):
            chunk_id = wid + c * NW

            @pl.when(chunk_id < n_chunks)
            def _():
                base = chunk_id * CHUNK
                pltpu.sync_copy(x_hbm.at[pl.ds(2 * base, 2 * CHUNK)], x_v)

                @plsc.parallel_loop(0, n_vec, unroll=8)
                def vec_body(v):
                    flat = v * (2 * L) + iota2
                    x0 = plsc.load_gather(x_v, [flat])
                    x1 = plsc.load_gather(x_v, [flat + 1])
                    p0 = _search_and_eval(x0, s0_v, a0_v, b0_v, c0_v, d0_v,
                                          bits)
                    p1 = _search_and_eval(x1, s1_v, a1_v, b1_v, c1_v, d1_v,
                                          bits)
                    out_v[pl.ds(v * L, L)] = p0 * p1
                pltpu.sync_copy(out_v, out_hbm.at[pl.ds(base, CHUNK)])

            return _

        lax.fori_loop(0, chunks_per_worker, chunk_body, None)

    return sc_kernel


def kernel(x, knots, a, b, c, d):
    n = x.shape[0]
    k = knots.shape[0]
    top_bit = 1 << (math.ceil(math.log2(k)) - 1)
    pad = 2 * top_bit - (k - 1)
    inf = jnp.full((pad,), jnp.inf, jnp.float32)
    s0 = jnp.concatenate([knots[:k - 1, 0], inf])
    s1 = jnp.concatenate([knots[:k - 1, 1], inf])

    def col(t, j):
        return jnp.concatenate([t[:, j], jnp.zeros((1,), jnp.float32)])

    sc = _make_sc_kernel(n, k)
    return sc(x.reshape(-1), s0, s1,
              col(a, 0), col(b, 0), col(c, 0), col(d, 0),
              col(a, 1), col(b, 1), col(c, 1), col(d, 1))
